# baseline (speedup 1.0000x reference)
import functools

import jax
import jax.numpy as jnp
from jax import lax
from jax.experimental import pallas as pl
from jax.experimental.pallas import tpu as pltpu
from jax.experimental.pallas import tpu_sc as plsc

_V = 100000
_VMAIN = 99968
_NCOLS = 781
_B = 16384
_LIST = 16400
_TRASH = 16390
_CPS = 9
_NR = 3
_PK = 16384


@functools.lru_cache(maxsize=None)
def _make_scan_gather():
    info = plsc.get_sparse_core_info()
    NC, NS = info.num_cores, info.num_subcores
    NW = NC * NS
    mesh = plsc.VectorSubcoreMesh(core_axis_name="c", subcore_axis_name="s")
    B = _B

    @functools.partial(
        pl.kernel,
        mesh=mesh,
        out_type=jax.ShapeDtypeStruct((B + NW, 128), jnp.float32),
        scratch_types=[
            pltpu.VMEM((4096,), jnp.int32),
            pltpu.VMEM((_LIST,), jnp.int32),
            pltpu.VMEM((_LIST,), jnp.int32),
            pltpu.VMEM((64, _CPS * 128), jnp.float32),
            pltpu.VMEM((128, 128), jnp.float32),
            pltpu.VMEM((2, 128), jnp.int32),
            pltpu.SemaphoreType.DMA,
            pltpu.SemaphoreType.DMA,
        ],
        compiler_params=pltpu.CompilerParams(
            use_tc_tiling_on_sc=True, needs_layout_passes=False
        ),
    )
    def scan_k(tT, tail_h, ids_h, out_h, ids_v, pk_list, pk_sub,
               chunk, rowbuf, slots2, sem, sem2):
        wid = lax.axis_index("s") * NC + lax.axis_index("c")
        c_lo = (wid * _NCOLS) // NW
        c_hi = ((wid + 1) * _NCOLS) // NW
        lo = c_lo * 128
        hi = c_hi * 128
        is_last = wid == (NW - 1)
        hi_f = jnp.where(is_last, _V, hi)
        lane = lax.iota(jnp.int32, 16)
        sent = B + wid
        zero16 = jnp.zeros((16,), jnp.int32)
        sent16 = zero16 + sent
        for k in range(8):
            slots2[0, pl.ds(k * 16, 16)] = sent16

        def fire(base):
            for j in range(_CPS):
                col = jnp.minimum(base + j, _NCOLS - 1)
                pltpu.async_copy(
                    tT.at[:, pl.ds(col * 128, 128)],
                    chunk.at[:, pl.ds(j * 128, 128)], sem)

        def drain():
            for j in range(_CPS):
                pltpu.make_async_copy(
                    tT.at[:, pl.ds(0, 128)],
                    chunk.at[:, pl.ds(j * 128, 128)], sem).wait()

        fire(c_lo)

        def fblk(blk, cnt0):
            pltpu.sync_copy(ids_h.at[pl.ds(blk * 4096, 4096)], ids_v)

            def fbody(g, cnt):
                for q in range(4):
                    u = ids_v[pl.ds(g * 64 + q * 16, 16)]
                    m = (u >= lo) & (u < hi_f)
                    cs = plsc.cumsum(m.astype(jnp.int32))
                    poss = jnp.where(m, cnt + cs - 1, _TRASH)
                    pk = u * _PK + (lane + (blk * 4096 + g * 64 + q * 16))
                    plsc.store_scatter(pk_list, [poss], pk)
                    cnt = cnt + plsc.all_reduce_population_count(m)
                return cnt

            return lax.fori_loop(0, 64, fbody, cnt0)

        cnt_v = lax.fori_loop(0, 4, fblk, zero16)
        cnt = jnp.max(cnt_v)
        ng = (cnt + 15) // 16

        def refilter(lo_r, hi_r):
            plo = lo_r * _PK
            phi = hi_r * _PK

            def sbody(g, cnt_s):
                valid = (lane + g * 16) < cnt
                pk = pk_list[pl.ds(g * 16, 16)]
                m = valid & (pk >= plo) & (pk < phi)
                cs = plsc.cumsum(m.astype(jnp.int32))
                poss = jnp.where(m, cnt_s + cs - 1, _TRASH)
                plsc.store_scatter(pk_sub, [poss], pk)
                return cnt_s + plsc.all_reduce_population_count(m)

            return jnp.max(lax.fori_loop(0, ng, sbody, zero16))

        def extract(cnt_s, ubase, buf, span, tail):
            ng_s = (cnt_s + 15) // 16

            def bbody(bi, _):
                def gbody(gg, _):
                    gb = bi * 128 + gg * 16
                    pk16 = pk_sub[pl.ds(gb, 16)]
                    u16 = lax.shift_right_logical(pk16, 14)
                    p16 = lax.bitwise_and(pk16, _PK - 1)
                    mv = (lane + gb) < cnt_s
                    e16 = lane + gg * 16
                    if tail:
                        ut = lax.clamp(jnp.int32(0), u16 - _VMAIN,
                                       jnp.int32(31))
                        row0 = lax.shift_right_logical(ut, 1)
                        col0 = lax.bitwise_and(ut, 1) * 64
                    else:
                        row0 = None
                        col0 = lax.clamp(jnp.int32(0), u16 - ubase,
                                         jnp.int32(span * 128 - 1))
                    for d in range(64):
                        dsp = jnp.full((16,), d, jnp.int32)
                        rowi = row0 if tail else dsp
                        coli = (col0 + d) if tail else col0
                        w = plsc.load_gather(buf, [rowi, coli])
                        plsc.store_scatter(rowbuf, [e16, dsp], w)
                    slots2[0, pl.ds(gg * 16, 16)] = jnp.where(
                        mv, p16, sent)
                    return 0

                ngrp = jnp.minimum(ng_s - bi * 8, 8)
                lax.fori_loop(0, ngrp, gbody, 0)
                pltpu.async_copy(rowbuf, out_h.at[slots2.at[0]],
                                 sem2).wait()
                return 0

            lax.fori_loop(0, (cnt_s + 127) // 128, bbody, 0)

        def rbody(r, _):
            c0 = c_lo + r * _CPS
            cnt_s = refilter(c0 * 128,
                             jnp.minimum(hi, (c0 + _CPS) * 128))
            drain()
            extract(cnt_s, c0 * 128, chunk, _CPS, False)

            @pl.when(r + 1 < _NR)
            def _():
                fire(c_lo + (r + 1) * _CPS)
            return 0

        lax.fori_loop(0, _NR, rbody, 0)

        pltpu.sync_copy(tail_h, chunk.at[pl.ds(0, 16), pl.ds(0, 128)])
        cnt_t = refilter(jnp.int32(_VMAIN), jnp.int32(_V))
        extract(cnt_t, 0, chunk, _CPS, True)

    return scan_k


def _mlp_body(x_ref, w1_ref, b1_ref, w2_ref, b2_ref, o_ref):
    x = x_ref[...][:, :64]
    h_t = lax.dot_general(
        w1_ref[...], x, (((1,), (1,)), ((), ())),
        preferred_element_type=jnp.float32,
    )
    h_t = jnp.maximum(h_t + b1_ref[...], 0.0)
    y_t = lax.dot_general(
        w2_ref[...], h_t, (((1,), (0,)), ((), ())),
        preferred_element_type=jnp.float32,
    )
    y_t = y_t + b2_ref[...]
    norm = jnp.sqrt(jnp.sum(y_t * y_t, axis=0, keepdims=True))
    o_ref[...] = y_t / jnp.maximum(norm, 1e-12)


@functools.lru_cache(maxsize=None)
def _make_mlp(B, D, H, blk):
    grid = (B // blk,)
    return pl.pallas_call(
        _mlp_body,
        grid=grid,
        in_specs=[
            pl.BlockSpec((blk, 128), lambda i: (i, 0)),
            pl.BlockSpec((H, D), lambda i: (0, 0)),
            pl.BlockSpec((H, 1), lambda i: (0, 0)),
            pl.BlockSpec((D, H), lambda i: (0, 0)),
            pl.BlockSpec((D, 1), lambda i: (0, 0)),
        ],
        out_specs=pl.BlockSpec((D, blk), lambda i: (0, i)),
        out_shape=jax.ShapeDtypeStruct((D, B), jnp.float32),
    )


def kernel(user_ids, table, W1, b1, W2, b2):
    V, D = table.shape
    H = W1.shape[0]
    B = user_ids.shape[0]
    ids = user_ids.astype(jnp.int32)
    tT = table.T
    tail2 = table[_VMAIN:].reshape(16, 2 * D)
    x2 = _make_scan_gather()(tT, tail2, ids)
    mlp = _make_mlp(B, D, H, 4096)
    out_t = mlp(x2, W1, b1.reshape(H, 1), W2, b2.reshape(D, 1))
    return out_t.T


# speedup vs baseline: 1.0067x; 1.0067x over previous
import functools

import jax
import jax.numpy as jnp
from jax import lax
from jax.experimental import pallas as pl
from jax.experimental.pallas import tpu as pltpu
from jax.experimental.pallas import tpu_sc as plsc

_V = 100000
_VMAIN = 99968
_NCOLS = 781
_B = 16384
_LIST = 16400
_TRASH = 16390
_CPS = 9
_NR = 3
_PK = 16384


@functools.lru_cache(maxsize=None)
def _make_scan_gather():
    info = plsc.get_sparse_core_info()
    NC, NS = info.num_cores, info.num_subcores
    NW = NC * NS
    mesh = plsc.VectorSubcoreMesh(core_axis_name="c", subcore_axis_name="s")
    B = _B

    @functools.partial(
        pl.kernel,
        mesh=mesh,
        out_type=jax.ShapeDtypeStruct((B + NW, 128), jnp.float32),
        scratch_types=[
            pltpu.VMEM((4096,), jnp.int32),
            pltpu.VMEM((_LIST,), jnp.int32),
            pltpu.VMEM((_LIST,), jnp.int32),
            pltpu.VMEM((64, _CPS * 128), jnp.float32),
            pltpu.VMEM((128, 128), jnp.float32),
            pltpu.VMEM((2, 128), jnp.int32),
            pltpu.SemaphoreType.DMA,
            pltpu.SemaphoreType.DMA,
        ],
        compiler_params=pltpu.CompilerParams(
            use_tc_tiling_on_sc=True, needs_layout_passes=False
        ),
    )
    def scan_k(tT, tail_h, ids_h, out_h, ids_v, pk_list, pk_sub,
               chunk, rowbuf, slots2, sem, sem2):
        wid = lax.axis_index("s") * NC + lax.axis_index("c")
        c_lo = (wid * _NCOLS) // NW
        c_hi = ((wid + 1) * _NCOLS) // NW
        lo = c_lo * 128
        hi = c_hi * 128
        is_last = wid == (NW - 1)
        hi_f = jnp.where(is_last, _V, hi)
        lane = lax.iota(jnp.int32, 16)
        sent = B + wid
        zero16 = jnp.zeros((16,), jnp.int32)
        sent16 = zero16 + sent
        for k in range(8):
            slots2[0, pl.ds(k * 16, 16)] = sent16

        def fblk(blk, cnt0):
            pltpu.sync_copy(ids_h.at[pl.ds(blk * 4096, 4096)], ids_v)

            def fbody(g, cnt):
                for q in range(4):
                    u = ids_v[pl.ds(g * 64 + q * 16, 16)]
                    m = (u >= lo) & (u < hi_f)
                    cs = plsc.cumsum(m.astype(jnp.int32))
                    poss = jnp.where(m, cnt + cs - 1, _TRASH)
                    pk = u * _PK + (lane + (blk * 4096 + g * 64 + q * 16))
                    plsc.store_scatter(pk_list, [poss], pk)
                    cnt = cnt + plsc.all_reduce_population_count(m)
                return cnt

            return lax.fori_loop(0, 64, fbody, cnt0)

        cnt_v = lax.fori_loop(0, 4, fblk, zero16)
        cnt = jnp.max(cnt_v)
        ng = (cnt + 15) // 16

        def refilter(lo_r, hi_r):
            plo = lo_r * _PK
            phi = hi_r * _PK

            def sbody(g, cnt_s):
                valid = (lane + g * 16) < cnt
                pk = pk_list[pl.ds(g * 16, 16)]
                m = valid & (pk >= plo) & (pk < phi)
                cs = plsc.cumsum(m.astype(jnp.int32))
                poss = jnp.where(m, cnt_s + cs - 1, _TRASH)
                plsc.store_scatter(pk_sub, [poss], pk)
                return cnt_s + plsc.all_reduce_population_count(m)

            return jnp.max(lax.fori_loop(0, ng, sbody, zero16))

        def extract(cnt_s, ubase, buf, span, tail):
            ng_s = (cnt_s + 15) // 16

            def bbody(bi, _):
                def gbody(gg, _):
                    gb = bi * 128 + gg * 16
                    pk16 = pk_sub[pl.ds(gb, 16)]
                    u16 = lax.shift_right_logical(pk16, 14)
                    p16 = lax.bitwise_and(pk16, _PK - 1)
                    mv = (lane + gb) < cnt_s
                    e16 = lane + gg * 16
                    if tail:
                        ut = lax.clamp(jnp.int32(0), u16 - _VMAIN,
                                       jnp.int32(31))
                        row0 = lax.shift_right_logical(ut, 1)
                        col0 = lax.bitwise_and(ut, 1) * 64
                    else:
                        row0 = None
                        col0 = lax.clamp(jnp.int32(0), u16 - ubase,
                                         jnp.int32(span * 128 - 1))
                    for d in range(64):
                        dsp = jnp.full((16,), d, jnp.int32)
                        rowi = row0 if tail else dsp
                        coli = (col0 + d) if tail else col0
                        w = plsc.load_gather(buf, [rowi, coli])
                        plsc.store_scatter(rowbuf, [e16, dsp], w)
                    slots2[0, pl.ds(gg * 16, 16)] = jnp.where(
                        mv, p16, sent)
                    return 0

                ngrp = jnp.minimum(ng_s - bi * 8, 8)
                lax.fori_loop(0, ngrp, gbody, 0)
                pltpu.async_copy(rowbuf, out_h.at[slots2.at[0]],
                                 sem2).wait()
                return 0

            lax.fori_loop(0, (cnt_s + 127) // 128, bbody, 0)

        def rbody(r, _):
            c0 = c_lo + r * _CPS
            copies = []
            for j in range(_CPS):
                col = jnp.minimum(c0 + j, _NCOLS - 1)
                copies.append(pltpu.async_copy(
                    tT.at[:, pl.ds(col * 128, 128)],
                    chunk.at[:, pl.ds(j * 128, 128)], sem))
            cnt_s = refilter(c0 * 128,
                             jnp.minimum(hi, (c0 + _CPS) * 128))
            for c in copies:
                c.wait()
            extract(cnt_s, c0 * 128, chunk, _CPS, False)
            return 0

        lax.fori_loop(0, _NR, rbody, 0)

        pltpu.sync_copy(tail_h, chunk.at[pl.ds(0, 16), pl.ds(0, 128)])
        cnt_t = refilter(jnp.int32(_VMAIN), jnp.int32(_V))
        extract(cnt_t, 0, chunk, _CPS, True)

    return scan_k


def _mlp_body(x_ref, w1_ref, b1_ref, w2_ref, b2_ref, o_ref):
    x = x_ref[...][:, :64]
    h_t = lax.dot_general(
        w1_ref[...], x, (((1,), (1,)), ((), ())),
        preferred_element_type=jnp.float32,
    )
    h_t = jnp.maximum(h_t + b1_ref[...], 0.0)
    y_t = lax.dot_general(
        w2_ref[...], h_t, (((1,), (0,)), ((), ())),
        preferred_element_type=jnp.float32,
    )
    y_t = y_t + b2_ref[...]
    norm = jnp.sqrt(jnp.sum(y_t * y_t, axis=0, keepdims=True))
    o_ref[...] = y_t / jnp.maximum(norm, 1e-12)


@functools.lru_cache(maxsize=None)
def _make_mlp(B, D, H, blk):
    grid = (B // blk,)
    return pl.pallas_call(
        _mlp_body,
        grid=grid,
        in_specs=[
            pl.BlockSpec((blk, 128), lambda i: (i, 0)),
            pl.BlockSpec((H, D), lambda i: (0, 0)),
            pl.BlockSpec((H, 1), lambda i: (0, 0)),
            pl.BlockSpec((D, H), lambda i: (0, 0)),
            pl.BlockSpec((D, 1), lambda i: (0, 0)),
        ],
        out_specs=pl.BlockSpec((D, blk), lambda i: (0, i)),
        out_shape=jax.ShapeDtypeStruct((D, B), jnp.float32),
    )


def kernel(user_ids, table, W1, b1, W2, b2):
    V, D = table.shape
    H = W1.shape[0]
    B = user_ids.shape[0]
    ids = user_ids.astype(jnp.int32)
    tT = table.T
    tail2 = table[_VMAIN:].reshape(16, 2 * D)
    x2 = _make_scan_gather()(tT, tail2, ids)
    mlp = _make_mlp(B, D, H, 4096)
    out_t = mlp(x2, W1, b1.reshape(H, 1), W2, b2.reshape(D, 1))
    return out_t.T
